# baseline (speedup 1.0000x reference)
import functools
import jax
import jax.numpy as jnp
from jax import lax
from jax.experimental import pallas as pl
from jax.experimental.pallas import tpu as pltpu
from jax.experimental.pallas import tpu_sc as plsc

N = 1024
KNN = 20
B = 8
NEG = -3.4e38


def _lrelu(v):
    return jnp.where(v >= 0, v, 0.2 * v)


def _matmul_nt(a, b):
    return lax.dot_general(a, b, (((1,), (1,)), ((), ())),
                           preferred_element_type=jnp.float32)


def _pair_of(xT):
    xx = jnp.sum(xT * xT, axis=1, keepdims=True)
    return 2.0 * _matmul_nt(xT, xT) - xx - jnp.transpose(xx)


def _pair0_body(xT_ref, pair_ref):
    pair_ref[0] = _pair_of(xT_ref[0])


def _pair0(xTp):
    return pl.pallas_call(
        _pair0_body,
        grid=(B,),
        in_specs=[pl.BlockSpec((1, N, 16), lambda i: (i, 0, 0))],
        out_specs=pl.BlockSpec((1, N, N), lambda i: (i, 0, 0)),
        out_shape=jax.ShapeDtypeStruct((B, N, N), jnp.float32),
    )(xTp)


def _stage_body(last, hcat_ref, W_ref, g_ref, b_ref, WaT_ref, WdT_ref,
                x_ref, pair_ref, ya_ref=None, dt_ref=None):
    k = pl.program_id(1)
    hk = hcat_ref[0, 0]
    h = _matmul_nt(hk, W_ref[...])
    a = _lrelu(h * g_ref[...] + b_ref[...])

    @pl.when(k == 0)
    def _():
        x_ref[0] = a

    @pl.when(k > 0)
    def _():
        x_ref[0] = jnp.maximum(x_ref[0], a)

    @pl.when(k == KNN - 1)
    def _():
        xT = x_ref[0]
        pair_ref[0] = _pair_of(xT)
        if last:
            ya_ref[0] = jnp.dot(xT, WaT_ref[...],
                                preferred_element_type=jnp.float32)
            dt_ref[0] = jnp.dot(xT, WdT_ref[...],
                                preferred_element_type=jnp.float32)


def _stage(last, C2, O, Onx, hcat, W, g, b, WaT, WdT):
    out_shapes = [
        jax.ShapeDtypeStruct((B, N, O), jnp.float32),
        jax.ShapeDtypeStruct((B, N, N), jnp.float32),
        jax.ShapeDtypeStruct((B, N, Onx), jnp.float32),
        jax.ShapeDtypeStruct((B, N, Onx), jnp.float32),
    ]
    out_specs = [
        pl.BlockSpec((1, N, O), lambda bi, ki: (bi, 0, 0)),
        pl.BlockSpec((1, N, N), lambda bi, ki: (bi, 0, 0)),
        pl.BlockSpec((1, N, Onx), lambda bi, ki: (bi, 0, 0)),
        pl.BlockSpec((1, N, Onx), lambda bi, ki: (bi, 0, 0)),
    ]
    if not last:
        out_shapes = out_shapes[:2]
        out_specs = out_specs[:2]
    Cw = WaT.shape[0]
    res = pl.pallas_call(
        functools.partial(_stage_body, last),
        grid=(B, KNN),
        in_specs=[
            pl.BlockSpec((1, 1, N, C2), lambda bi, ki: (ki, bi, 0, 0)),
            pl.BlockSpec((O, C2), lambda bi, ki: (0, 0)),
            pl.BlockSpec((1, O), lambda bi, ki: (0, 0)),
            pl.BlockSpec((1, O), lambda bi, ki: (0, 0)),
            pl.BlockSpec((Cw, Onx), lambda bi, ki: (0, 0)),
            pl.BlockSpec((Cw, Onx), lambda bi, ki: (0, 0)),
        ],
        out_specs=out_specs,
        out_shape=out_shapes,
    )(hcat, W, g, b, WaT, WdT)
    if last:
        return res
    return res[0], res[1], None, None


def _head_body(x1_ref, x2_ref, x3_ref, nmax_ref, dT_ref, g4_ref, b4_ref,
               W5a_ref, W5b_ref, W5c_ref, W5d_ref, g5_ref, b5_ref,
               Wl1a_ref, Wl1b_ref, gl1_ref, bl1_ref,
               Wl2_ref, lb2_ref, gl2_ref, bl2_ref,
               Wl3_ref, lb3_ref, out_ref):
    x4 = _lrelu((nmax_ref[0] + dT_ref[0]) * g4_ref[...] + b4_ref[...])
    h = (_matmul_nt(x1_ref[0], W5a_ref[...]) +
         _matmul_nt(x2_ref[0], W5b_ref[...]) +
         _matmul_nt(x3_ref[0], W5c_ref[...]) +
         _matmul_nt(x4, W5d_ref[...]))
    h = _lrelu(h * g5_ref[...] + b5_ref[...])
    amp = jnp.max(h, axis=0, keepdims=True)
    aap = jnp.sum(h, axis=0, keepdims=True) / N
    f = _matmul_nt(amp, Wl1a_ref[...]) + _matmul_nt(aap, Wl1b_ref[...])
    f = _lrelu(f * gl1_ref[...] + bl1_ref[...])
    f = _matmul_nt(f, Wl2_ref[...]) + lb2_ref[...]
    f = _lrelu(f * gl2_ref[...] + bl2_ref[...])
    out_ref[0] = _matmul_nt(f, Wl3_ref[...]) + lb3_ref[...]


def _head(x1T, x2T, x3T, nmax4T, d4T, g4, b4, W5, g5, b5,
          Wl1, gl1, bl1, Wl2, lb2, gl2, bl2, Wl3, lb3):
    W5a = W5[:, :64]
    W5b = W5[:, 64:128]
    W5c = W5[:, 128:256]
    W5d = W5[:, 256:512]
    Wl1a = Wl1[:, :1024]
    Wl1b = Wl1[:, 1024:]

    def bs(shape):
        nd = len(shape)
        return pl.BlockSpec(shape, lambda i: (0,) * nd)

    bsx = lambda c: pl.BlockSpec((1, N, c), lambda i: (i, 0, 0))
    return pl.pallas_call(
        _head_body,
        grid=(B,),
        in_specs=[bsx(64), bsx(64), bsx(128), bsx(256), bsx(256),
                  bs((1, 256)), bs((1, 256)),
                  bs((1024, 64)), bs((1024, 64)), bs((1024, 128)), bs((1024, 256)),
                  bs((1, 1024)), bs((1, 1024)),
                  bs((512, 1024)), bs((512, 1024)), bs((1, 512)), bs((1, 512)),
                  bs((256, 512)), bs((1, 256)), bs((1, 256)), bs((1, 256)),
                  bs((40, 256)), bs((1, 40))],
        out_specs=pl.BlockSpec((1, 1, 40), lambda i: (i, 0, 0)),
        out_shape=jax.ShapeDtypeStruct((B, 1, 40), jnp.float32),
    )(x1T, x2T, x3T, nmax4T, d4T, g4, b4, W5a, W5b, W5c, W5d,
      g5, b5, Wl1a, Wl1b, gl1, bl1, Wl2, lb2, gl2, bl2, Wl3, lb3)[:, 0, :]


def _topk_phases(prow8, q8, iota):
    qv = jnp.full((16,), q8, jnp.int32)
    gm = jnp.full((16,), NEG, jnp.float32)
    for g in range(16):
        v0 = plsc.load_gather(prow8, [qv, g * 64 + iota])
        v1 = plsc.load_gather(prow8, [qv, g * 64 + 16 + iota])
        v2 = plsc.load_gather(prow8, [qv, g * 64 + 32 + iota])
        v3 = plsc.load_gather(prow8, [qv, g * 64 + 48 + iota])
        m = jnp.maximum(jnp.maximum(v0, v1), jnp.maximum(v2, v3))
        gm = jnp.where(iota == g, jnp.max(m), gm)
    positions = []
    for j in range(KNN):
        t = jnp.max(gm)
        gsel = jnp.min(jnp.where(gm == t, iota, 1000))
        gbase = gsel * 64
        vs = []
        cands = []
        for i in range(4):
            gi = gbase + i * 16 + iota
            v = plsc.load_gather(prow8, [qv, gi])
            vs.append(v)
            cands.append(jnp.where(v == t, i * 16 + iota, 2000))
        pos_rel = jnp.min(jnp.minimum(
            jnp.minimum(cands[0], cands[1]),
            jnp.minimum(cands[2], cands[3])))
        nmx = NEG
        for i in range(4):
            nv = jnp.where(i * 16 + iota == pos_rel, NEG, vs[i])
            plsc.store_scatter(prow8, [qv, gbase + i * 16 + iota], nv)
            nmx = jnp.maximum(nmx, jnp.max(nv))
        gm = jnp.where(iota == gsel, nmx, gm)
        positions.append(gbase + pos_rel)
    return positions


def _sc_hcat(Cp, Cin):
    ROWS = B * N
    NW = 32
    PW = ROWS // NW
    W2C = max(16, 2 * Cin)
    Q = 128 // W2C
    TR = N * Cp // 128
    mesh = plsc.VectorSubcoreMesh(core_axis_name="c", subcore_axis_name="s")
    out_t = jax.ShapeDtypeStruct((KNN * ROWS * W2C // 128, 128), jnp.float32)

    QB = 8 * Q

    @functools.partial(
        pl.kernel, mesh=mesh, out_type=out_t,
        compiler_params=pltpu.CompilerParams(needs_layout_passes=False),
        scratch_types=[
            pltpu.VMEM((8, N), jnp.float32),
            pltpu.VMEM((TR, 128), jnp.float32),
            pltpu.VMEM((8, 16), jnp.float32),
            pltpu.VMEM((KNN, 8, 128), jnp.float32),
            pltpu.SemaphoreType.DMA,
        ])
    def k(pair_hbm, table_hbm, out_hbm, prow8, tbl, cent, hst, sem):
        cid = lax.axis_index("c")
        sid = lax.axis_index("s")
        wid = sid * 2 + cid
        base = wid * PW
        boff = (base // N) * N
        iota = lax.iota(jnp.int32, 16)
        pltpu.sync_copy(
            table_hbm.at[pl.ds(pl.multiple_of((boff * Cp) // 128, 8), TR)],
            tbl)

        def tload(flat):
            return plsc.load_gather(tbl, [flat >> 7, flat & 127])

        def batch_body(bi, carry):
            row0 = base + bi * QB

            def sub_body(sq, carry1):
                r8 = row0 + sq * 8
                pltpu.sync_copy(
                    pair_hbm.at[pl.ds(pl.multiple_of(r8, 8), 8)], prow8)

                @plsc.parallel_loop(0, 8, unroll=2)
                def point_body(q8):
                    row = r8 + q8
                    nloc = row - boff
                    qv = jnp.full((16,), q8, jnp.int32)
                    ccs = [tload(nloc * Cp + c * 16 + iota)
                           for c in range(Cp // 16)]
                    if Cin < 16:
                        plsc.store_scatter(cent, [qv, iota], ccs[0])
                        csh = plsc.load_gather(
                            cent, [qv, jnp.maximum(iota - Cin, 0)])
                    positions = _topk_phases(prow8, q8, iota)
                    for j, pos in enumerate(positions):
                        jv = jnp.full((16,), j, jnp.int32)
                        if Cin < 16:
                            v = tload(pos * Cp + iota)
                            d = v - ccs[0]
                            hrow = jnp.where(
                                iota < Cin, d,
                                jnp.where(iota < 2 * Cin, csh, 0.0))
                            plsc.store_scatter(
                                hst, [jv, jnp.full((16,), sq, jnp.int32),
                                      q8 * 16 + iota], hrow)
                        else:
                            for c in range(Cin // 16):
                                v = tload(pos * Cp + c * 16 + iota)
                                plsc.store_scatter(
                                    hst, [jv, qv, c * 16 + iota], v - ccs[c])
                                plsc.store_scatter(
                                    hst, [jv, qv, Cin + c * 16 + iota],
                                    ccs[c])
                return carry1

            lax.fori_loop(0, QB // 8, sub_body, 0)
            dmas = [pltpu.async_copy(
                hst.at[kk],
                out_hbm.at[pl.ds(
                    pl.multiple_of(((kk * ROWS + row0) * W2C) // 128, 8), 8)],
                sem) for kk in range(KNN)]
            for d in dmas:
                d.wait()
            return carry

        lax.fori_loop(0, PW // QB, batch_body, 0)

    return k


def _sc_max(Cp):
    ROWS = B * N
    NW = 32
    PW = ROWS // NW
    mesh = plsc.VectorSubcoreMesh(core_axis_name="c", subcore_axis_name="s")
    out_t = jax.ShapeDtypeStruct((ROWS, Cp), jnp.float32)

    @functools.partial(
        pl.kernel, mesh=mesh, out_type=out_t,
        compiler_params=pltpu.CompilerParams(needs_layout_passes=False),
        scratch_types=[
            pltpu.VMEM((8, N), jnp.float32),
            pltpu.VMEM((8, 32), jnp.int32),
            pltpu.VMEM((8, 32, Cp), jnp.float32),
            pltpu.VMEM((8, Cp), jnp.float32),
            pltpu.SemaphoreType.DMA,
        ])
    def k(pair_hbm, table_hbm, out_hbm, prow8, idxb, rowsb, hst, sem):
        cid = lax.axis_index("c")
        sid = lax.axis_index("s")
        wid = sid * 2 + cid
        base = wid * PW
        iota = lax.iota(jnp.int32, 16)

        def batch_body(bi, carry):
            row0 = base + bi * 8
            pltpu.sync_copy(
                pair_hbm.at[pl.ds(pl.multiple_of(row0, 8), 8)], prow8)

            @plsc.parallel_loop(0, 8, unroll=2)
            def point_topk(q8):
                row = row0 + q8
                boff = (row // N) * N
                qv = jnp.full((16,), q8, jnp.int32)
                positions = _topk_phases(prow8, q8, iota)
                idxlo = jnp.full((16,), row, jnp.int32)
                idxhi = jnp.full((16,), row, jnp.int32)
                for j, pos in enumerate(positions):
                    g = boff + pos
                    if j < 16:
                        idxlo = jnp.where(iota == j, g, idxlo)
                    else:
                        idxhi = jnp.where(iota == (j - 16), g, idxhi)
                plsc.store_scatter(idxb, [qv, iota], idxlo)
                plsc.store_scatter(idxb, [qv, 16 + iota], idxhi)

            dmas = [pltpu.async_copy(table_hbm.at[idxb.at[p]], rowsb.at[p],
                                     sem) for p in range(8)]
            for d in dmas:
                d.wait()

            @plsc.parallel_loop(0, 8, unroll=2)
            def point_max(q8):
                qv = jnp.full((16,), q8, jnp.int32)
                for c in range(Cp // 16):
                    m = plsc.load_gather(
                        rowsb, [qv, jnp.zeros((16,), jnp.int32),
                                c * 16 + iota])
                    for kk in range(1, KNN):
                        m = jnp.maximum(m, plsc.load_gather(
                            rowsb, [qv, jnp.full((16,), kk, jnp.int32),
                                    c * 16 + iota]))
                    plsc.store_scatter(hst, [qv, c * 16 + iota], m)

            pltpu.sync_copy(
                hst, out_hbm.at[pl.ds(pl.multiple_of(row0, 8), 8)])
            return carry

        lax.fori_loop(0, PW // 8, batch_body, 0)

    return k


def kernel(x, W1, g1, b1, W2, g2, b2, W3, g3, b3, W4, g4, b4, W5, g5, b5,
           Wl1, gl1, bl1, Wl2, lb2, gl2, bl2, Wl3, lb3):
    xT = jnp.swapaxes(x, 1, 2)
    xTp = jnp.pad(xT, ((0, 0), (0, 0), (0, 13)))
    r2 = lambda v: v.reshape(1, -1)
    W1p = jnp.pad(W1, ((0, 0), (0, 10)))
    Wa4T = jnp.transpose(W4[:, :128])
    Wd4T = jnp.transpose(W4[:, 128:] - W4[:, :128])
    zW = jnp.zeros((1, 1), jnp.float32)

    flat = lambda t: t.reshape(B * N, -1)
    t2 = lambda t: t.reshape(-1, 128)
    hshape = lambda w: (KNN, B, N, w)

    pair = _pair0(xTp)
    hc = _sc_hcat(16, 3)(flat(pair), t2(xTp))
    x1, pair, _, _ = _stage(False, 16, 64, 1, hc.reshape(hshape(16)),
                            W1p, r2(g1), r2(b1), zW, zW)
    hc = _sc_hcat(64, 64)(flat(pair), t2(x1))
    x2, pair, _, _ = _stage(False, 128, 64, 1, hc.reshape(hshape(128)),
                            W2, r2(g2), r2(b2), zW, zW)
    hc = _sc_hcat(64, 64)(flat(pair), t2(x2))
    x3, pair, ya4, dT4 = _stage(True, 128, 128, 256, hc.reshape(hshape(128)),
                                W3, r2(g3), r2(b3), Wa4T, Wd4T)
    nm4 = _sc_max(256)(flat(pair), flat(ya4)).reshape(B, N, 256)
    return _head(x1, x2, x3, nm4, dT4, r2(g4), r2(b4), W5, r2(g5), r2(b5),
                 Wl1, r2(gl1), r2(bl1), Wl2, r2(lb2), r2(gl2), r2(bl2),
                 Wl3, r2(lb3))


# speedup vs baseline: 1.1300x; 1.1300x over previous
import functools
import jax
import jax.numpy as jnp
from jax import lax
from jax.experimental import pallas as pl
from jax.experimental.pallas import tpu as pltpu
from jax.experimental.pallas import tpu_sc as plsc

N = 1024
KNN = 20
B = 8
NEG = -3.4e38


def _lrelu(v):
    return jnp.where(v >= 0, v, 0.2 * v)


def _matmul_nt(a, b):
    return lax.dot_general(a, b, (((1,), (1,)), ((), ())),
                           preferred_element_type=jnp.float32)


def _pair_of(xT):
    xx = jnp.sum(xT * xT, axis=1, keepdims=True)
    return 2.0 * _matmul_nt(xT, xT) - xx - jnp.transpose(xx)


def _pair0_body(xT_ref, pair_ref):
    pair_ref[0] = _pair_of(xT_ref[0])


def _pair0(xTp):
    return pl.pallas_call(
        _pair0_body,
        grid=(B,),
        in_specs=[pl.BlockSpec((1, N, 16), lambda i: (i, 0, 0))],
        out_specs=pl.BlockSpec((1, N, N), lambda i: (i, 0, 0)),
        out_shape=jax.ShapeDtypeStruct((B, N, N), jnp.float32),
    )(xTp)


def _stage_body(last, hcat_ref, W_ref, g_ref, b_ref, WaT_ref, WdT_ref,
                x_ref, pair_ref, ya_ref=None, dt_ref=None):
    k = pl.program_id(1)
    hk = hcat_ref[0, 0]
    h = _matmul_nt(hk, W_ref[...])
    a = _lrelu(h * g_ref[...] + b_ref[...])

    @pl.when(k == 0)
    def _():
        x_ref[0] = a

    @pl.when(k > 0)
    def _():
        x_ref[0] = jnp.maximum(x_ref[0], a)

    @pl.when(k == KNN - 1)
    def _():
        xT = x_ref[0]
        pair_ref[0] = _pair_of(xT)
        if last:
            ya_ref[0] = jnp.dot(xT, WaT_ref[...],
                                preferred_element_type=jnp.float32)
            dt_ref[0] = jnp.dot(xT, WdT_ref[...],
                                preferred_element_type=jnp.float32)


def _stage(last, C2, O, Onx, hcat, W, g, b, WaT, WdT):
    out_shapes = [
        jax.ShapeDtypeStruct((B, N, O), jnp.float32),
        jax.ShapeDtypeStruct((B, N, N), jnp.float32),
        jax.ShapeDtypeStruct((B, N, Onx), jnp.float32),
        jax.ShapeDtypeStruct((B, N, Onx), jnp.float32),
    ]
    out_specs = [
        pl.BlockSpec((1, N, O), lambda bi, ki: (bi, 0, 0)),
        pl.BlockSpec((1, N, N), lambda bi, ki: (bi, 0, 0)),
        pl.BlockSpec((1, N, Onx), lambda bi, ki: (bi, 0, 0)),
        pl.BlockSpec((1, N, Onx), lambda bi, ki: (bi, 0, 0)),
    ]
    if not last:
        out_shapes = out_shapes[:2]
        out_specs = out_specs[:2]
    Cw = WaT.shape[0]
    res = pl.pallas_call(
        functools.partial(_stage_body, last),
        grid=(B, KNN),
        in_specs=[
            pl.BlockSpec((1, 1, N, C2), lambda bi, ki: (ki, bi, 0, 0)),
            pl.BlockSpec((O, C2), lambda bi, ki: (0, 0)),
            pl.BlockSpec((1, O), lambda bi, ki: (0, 0)),
            pl.BlockSpec((1, O), lambda bi, ki: (0, 0)),
            pl.BlockSpec((Cw, Onx), lambda bi, ki: (0, 0)),
            pl.BlockSpec((Cw, Onx), lambda bi, ki: (0, 0)),
        ],
        out_specs=out_specs,
        out_shape=out_shapes,
    )(hcat, W, g, b, WaT, WdT)
    if last:
        return res
    return res[0], res[1], None, None


def _head_body(x1_ref, x2_ref, x3_ref, nmax_ref, dT_ref, g4_ref, b4_ref,
               W5a_ref, W5b_ref, W5c_ref, W5d_ref, g5_ref, b5_ref,
               Wl1a_ref, Wl1b_ref, gl1_ref, bl1_ref,
               Wl2_ref, lb2_ref, gl2_ref, bl2_ref,
               Wl3_ref, lb3_ref, out_ref):
    x4 = _lrelu((nmax_ref[0] + dT_ref[0]) * g4_ref[...] + b4_ref[...])
    h = (_matmul_nt(x1_ref[0], W5a_ref[...]) +
         _matmul_nt(x2_ref[0], W5b_ref[...]) +
         _matmul_nt(x3_ref[0], W5c_ref[...]) +
         _matmul_nt(x4, W5d_ref[...]))
    h = _lrelu(h * g5_ref[...] + b5_ref[...])
    amp = jnp.max(h, axis=0, keepdims=True)
    aap = jnp.sum(h, axis=0, keepdims=True) / N
    f = _matmul_nt(amp, Wl1a_ref[...]) + _matmul_nt(aap, Wl1b_ref[...])
    f = _lrelu(f * gl1_ref[...] + bl1_ref[...])
    f = _matmul_nt(f, Wl2_ref[...]) + lb2_ref[...]
    f = _lrelu(f * gl2_ref[...] + bl2_ref[...])
    out_ref[0] = _matmul_nt(f, Wl3_ref[...]) + lb3_ref[...]


def _head(x1T, x2T, x3T, nmax4T, d4T, g4, b4, W5, g5, b5,
          Wl1, gl1, bl1, Wl2, lb2, gl2, bl2, Wl3, lb3):
    W5a = W5[:, :64]
    W5b = W5[:, 64:128]
    W5c = W5[:, 128:256]
    W5d = W5[:, 256:512]
    Wl1a = Wl1[:, :1024]
    Wl1b = Wl1[:, 1024:]

    def bs(shape):
        nd = len(shape)
        return pl.BlockSpec(shape, lambda i: (0,) * nd)

    bsx = lambda c: pl.BlockSpec((1, N, c), lambda i: (i, 0, 0))
    return pl.pallas_call(
        _head_body,
        grid=(B,),
        in_specs=[bsx(64), bsx(64), bsx(128), bsx(256), bsx(256),
                  bs((1, 256)), bs((1, 256)),
                  bs((1024, 64)), bs((1024, 64)), bs((1024, 128)), bs((1024, 256)),
                  bs((1, 1024)), bs((1, 1024)),
                  bs((512, 1024)), bs((512, 1024)), bs((1, 512)), bs((1, 512)),
                  bs((256, 512)), bs((1, 256)), bs((1, 256)), bs((1, 256)),
                  bs((40, 256)), bs((1, 40))],
        out_specs=pl.BlockSpec((1, 1, 40), lambda i: (i, 0, 0)),
        out_shape=jax.ShapeDtypeStruct((B, 1, 40), jnp.float32),
    )(x1T, x2T, x3T, nmax4T, d4T, g4, b4, W5a, W5b, W5c, W5d,
      g5, b5, Wl1a, Wl1b, gl1, bl1, Wl2, lb2, gl2, bl2, Wl3, lb3)[:, 0, :]


def _topk_phases(prow8, q8, iota):
    qv = jnp.full((16,), q8, jnp.int32)
    gm = jnp.full((16,), NEG, jnp.float32)
    for g in range(16):
        v0 = plsc.load_gather(prow8, [qv, g * 64 + iota])
        v1 = plsc.load_gather(prow8, [qv, g * 64 + 16 + iota])
        v2 = plsc.load_gather(prow8, [qv, g * 64 + 32 + iota])
        v3 = plsc.load_gather(prow8, [qv, g * 64 + 48 + iota])
        m = jnp.maximum(jnp.maximum(v0, v1), jnp.maximum(v2, v3))
        gm = jnp.where(iota == g, jnp.max(m), gm)
    positions = []
    for j in range(KNN):
        t = jnp.max(gm)
        gsel = jnp.min(jnp.where(gm == t, iota, 1000))
        gbase = gsel * 64
        vs = []
        cands = []
        for i in range(4):
            gi = gbase + i * 16 + iota
            v = plsc.load_gather(prow8, [qv, gi])
            vs.append(v)
            cands.append(jnp.where(v == t, i * 16 + iota, 2000))
        pos_rel = jnp.min(jnp.minimum(
            jnp.minimum(cands[0], cands[1]),
            jnp.minimum(cands[2], cands[3])))
        nmx = NEG
        for i in range(4):
            nv = jnp.where(i * 16 + iota == pos_rel, NEG, vs[i])
            plsc.store_scatter(prow8, [qv, gbase + i * 16 + iota], nv)
            nmx = jnp.maximum(nmx, jnp.max(nv))
        gm = jnp.where(iota == gsel, nmx, gm)
        positions.append(gbase + pos_rel)
    return positions


def _sc_hcat(Cp, Cin):
    ROWS = B * N
    NW = 32
    PW = ROWS // NW
    W2C = max(16, 2 * Cin)
    Q = 128 // W2C
    TR = N * Cp // 128
    mesh = plsc.VectorSubcoreMesh(core_axis_name="c", subcore_axis_name="s")
    out_t = jax.ShapeDtypeStruct((KNN * ROWS * W2C // 128, 128), jnp.float32)

    QB = 8 * Q

    @functools.partial(
        pl.kernel, mesh=mesh, out_type=out_t,
        compiler_params=pltpu.CompilerParams(needs_layout_passes=False),
        scratch_types=[
            pltpu.VMEM((8, N), jnp.float32),
            pltpu.VMEM((TR, 128), jnp.float32),
            pltpu.VMEM((8, 16), jnp.float32),
            pltpu.VMEM((KNN, 8, 128), jnp.float32),
            pltpu.SemaphoreType.DMA,
        ])
    def k(pair_hbm, table_hbm, out_hbm, prow8, tbl, cent, hst, sem):
        cid = lax.axis_index("c")
        sid = lax.axis_index("s")
        wid = sid * 2 + cid
        base = wid * PW
        boff = (base // N) * N
        iota = lax.iota(jnp.int32, 16)
        pltpu.sync_copy(
            table_hbm.at[pl.ds(pl.multiple_of((boff * Cp) // 128, 8), TR)],
            tbl)

        def tload(flat):
            return plsc.load_gather(tbl, [flat >> 7, flat & 127])

        def batch_body(bi, carry):
            row0 = base + bi * QB

            def sub_body(sq, carry1):
                r8 = row0 + sq * 8
                pltpu.sync_copy(
                    pair_hbm.at[pl.ds(pl.multiple_of(r8, 8), 8)], prow8)

                def point_body(q8, carry2):
                    row = r8 + q8
                    nloc = row - boff
                    qv = jnp.full((16,), q8, jnp.int32)
                    ccs = [tload(nloc * Cp + c * 16 + iota)
                           for c in range(Cp // 16)]
                    if Cin < 16:
                        plsc.store_scatter(cent, [qv, iota], ccs[0])
                        csh = plsc.load_gather(
                            cent, [qv, jnp.maximum(iota - Cin, 0)])
                    positions = _topk_phases(prow8, q8, iota)
                    for j, pos in enumerate(positions):
                        jv = jnp.full((16,), j, jnp.int32)
                        if Cin < 16:
                            v = tload(pos * Cp + iota)
                            d = v - ccs[0]
                            hrow = jnp.where(
                                iota < Cin, d,
                                jnp.where(iota < 2 * Cin, csh, 0.0))
                            plsc.store_scatter(
                                hst, [jv, jnp.full((16,), sq, jnp.int32),
                                      q8 * 16 + iota], hrow)
                        else:
                            for c in range(Cin // 16):
                                v = tload(pos * Cp + c * 16 + iota)
                                plsc.store_scatter(
                                    hst, [jv, qv, c * 16 + iota], v - ccs[c])
                                plsc.store_scatter(
                                    hst, [jv, qv, Cin + c * 16 + iota],
                                    ccs[c])
                    return carry2

                lax.fori_loop(0, 8, point_body, 0)
                return carry1

            lax.fori_loop(0, QB // 8, sub_body, 0)
            dmas = [pltpu.async_copy(
                hst.at[kk],
                out_hbm.at[pl.ds(
                    pl.multiple_of(((kk * ROWS + row0) * W2C) // 128, 8), 8)],
                sem) for kk in range(KNN)]
            for d in dmas:
                d.wait()
            return carry

        lax.fori_loop(0, PW // QB, batch_body, 0)

    return k


def _sc_max(Cp):
    ROWS = B * N
    NW = 32
    PW = ROWS // NW
    mesh = plsc.VectorSubcoreMesh(core_axis_name="c", subcore_axis_name="s")
    out_t = jax.ShapeDtypeStruct((ROWS, Cp), jnp.float32)

    @functools.partial(
        pl.kernel, mesh=mesh, out_type=out_t,
        compiler_params=pltpu.CompilerParams(needs_layout_passes=False),
        scratch_types=[
            pltpu.VMEM((8, N), jnp.float32),
            pltpu.VMEM((8, 32), jnp.int32),
            pltpu.VMEM((8, 32, Cp), jnp.float32),
            pltpu.VMEM((8, Cp), jnp.float32),
            pltpu.SemaphoreType.DMA,
        ])
    def k(pair_hbm, table_hbm, out_hbm, prow8, idxb, rowsb, hst, sem):
        cid = lax.axis_index("c")
        sid = lax.axis_index("s")
        wid = sid * 2 + cid
        base = wid * PW
        iota = lax.iota(jnp.int32, 16)

        def batch_body(bi, carry):
            row0 = base + bi * 8
            pltpu.sync_copy(
                pair_hbm.at[pl.ds(pl.multiple_of(row0, 8), 8)], prow8)

            @plsc.parallel_loop(0, 8, unroll=2)
            def point_topk(q8):
                row = row0 + q8
                boff = (row // N) * N
                qv = jnp.full((16,), q8, jnp.int32)
                positions = _topk_phases(prow8, q8, iota)
                idxlo = jnp.full((16,), row, jnp.int32)
                idxhi = jnp.full((16,), row, jnp.int32)
                for j, pos in enumerate(positions):
                    g = boff + pos
                    if j < 16:
                        idxlo = jnp.where(iota == j, g, idxlo)
                    else:
                        idxhi = jnp.where(iota == (j - 16), g, idxhi)
                plsc.store_scatter(idxb, [qv, iota], idxlo)
                plsc.store_scatter(idxb, [qv, 16 + iota], idxhi)

            dmas = [pltpu.async_copy(table_hbm.at[idxb.at[p]], rowsb.at[p],
                                     sem) for p in range(8)]
            for d in dmas:
                d.wait()

            @plsc.parallel_loop(0, 8, unroll=2)
            def point_max(q8):
                qv = jnp.full((16,), q8, jnp.int32)
                for c in range(Cp // 16):
                    m = plsc.load_gather(
                        rowsb, [qv, jnp.zeros((16,), jnp.int32),
                                c * 16 + iota])
                    for kk in range(1, KNN):
                        m = jnp.maximum(m, plsc.load_gather(
                            rowsb, [qv, jnp.full((16,), kk, jnp.int32),
                                    c * 16 + iota]))
                    plsc.store_scatter(hst, [qv, c * 16 + iota], m)

            pltpu.sync_copy(
                hst, out_hbm.at[pl.ds(pl.multiple_of(row0, 8), 8)])
            return carry

        lax.fori_loop(0, PW // 8, batch_body, 0)

    return k


def kernel(x, W1, g1, b1, W2, g2, b2, W3, g3, b3, W4, g4, b4, W5, g5, b5,
           Wl1, gl1, bl1, Wl2, lb2, gl2, bl2, Wl3, lb3):
    xT = jnp.swapaxes(x, 1, 2)
    xTp = jnp.pad(xT, ((0, 0), (0, 0), (0, 13)))
    r2 = lambda v: v.reshape(1, -1)
    W1p = jnp.pad(W1, ((0, 0), (0, 10)))
    Wa4T = jnp.transpose(W4[:, :128])
    Wd4T = jnp.transpose(W4[:, 128:] - W4[:, :128])
    zW = jnp.zeros((1, 1), jnp.float32)

    flat = lambda t: t.reshape(B * N, -1)
    t2 = lambda t: t.reshape(-1, 128)
    hshape = lambda w: (KNN, B, N, w)

    pair = _pair0(xTp)
    hc = _sc_hcat(16, 3)(flat(pair), t2(xTp))
    x1, pair, _, _ = _stage(False, 16, 64, 1, hc.reshape(hshape(16)),
                            W1p, r2(g1), r2(b1), zW, zW)
    hc = _sc_hcat(64, 64)(flat(pair), t2(x1))
    x2, pair, _, _ = _stage(False, 128, 64, 1, hc.reshape(hshape(128)),
                            W2, r2(g2), r2(b2), zW, zW)
    hc = _sc_hcat(64, 64)(flat(pair), t2(x2))
    x3, pair, ya4, dT4 = _stage(True, 128, 128, 256, hc.reshape(hshape(128)),
                                W3, r2(g3), r2(b3), Wa4T, Wd4T)
    nm4 = _sc_max(256)(flat(pair), flat(ya4)).reshape(B, N, 256)
    return _head(x1, x2, x3, nm4, dT4, r2(g4), r2(b4), W5, r2(g5), r2(b5),
                 Wl1, r2(gl1), r2(bl1), Wl2, r2(lb2), r2(gl2), r2(bl2),
                 Wl3, r2(lb3))
